# baseline (speedup 1.0000x reference)
import functools

import jax
import jax.numpy as jnp
import numpy as np
from jax import lax
from jax.experimental import pallas as pl
from jax.experimental.pallas import tpu as pltpu
from jax.experimental.pallas import tpu_sc as plsc

NC = 2
NS = 16
NW = NC * NS
DEG = 32
D = 128
DPK = D // 2
WG = DPK // 16
GROW = 128
NODES_PER_CHUNK = GROW // DEG
NBUF = 4


def _agg_body(b_per_w, nchunk, features, idx_all, out, idx_v, acc_v,
              *scratch):
    bufs = scratch[:NBUF]
    gsems = scratch[NBUF:2 * NBUF]

    wid = lax.axis_index("s") * NC + lax.axis_index("c")
    pltpu.sync_copy(idx_all.at[wid], idx_v)

    def fire_gather(c, b):
        pltpu.async_copy(features.at[idx_v.at[c]], bufs[b], gsems[b])

    def wait_gather(c, b):
        pltpu.make_async_copy(features.at[idx_v.at[c]], bufs[b],
                              gsems[b]).wait()

    hi_mask = jnp.full((16,), np.int32(np.uint32(0xFFFF0000).view(np.int32)),
                       jnp.int32)

    def compute_chunk(c, b):
        buf = bufs[b]

        def node_body(n, carry):
            row0 = n * DEG
            for g in range(WG):
                aL = jnp.zeros((16,), jnp.float32)
                aH = jnp.zeros((16,), jnp.float32)
                for j in range(DEG):
                    w = lax.bitcast_convert_type(
                        buf[row0 + j, pl.ds(g * 16, 16)], jnp.int32)
                    lo = lax.bitcast_convert_type(
                        lax.shift_left(w, 16), jnp.float32)
                    hi = lax.bitcast_convert_type(w & hi_mask, jnp.float32)
                    aL = aL + lo
                    aH = aH + hi
                col = c * NODES_PER_CHUNK + n
                acc_v[col, pl.ds(g * 32, 16)] = aL
                acc_v[col, pl.ds(g * 32 + 16, 16)] = aH
            return carry
        lax.fori_loop(0, NODES_PER_CHUNK, node_body, 0)

    for b in range(NBUF):
        fire_gather(b, b)

    def group_body(g, carry):
        for b in range(NBUF):
            i = g * NBUF + b
            wait_gather(i, b)
            compute_chunk(i, b)

            @pl.when(i + NBUF < nchunk)
            def _():
                fire_gather(i + NBUF, b)

        return carry

    lax.fori_loop(0, nchunk // NBUF, group_body, 0)
    pltpu.sync_copy(acc_v, out.at[pl.ds(wid * b_per_w, b_per_w)])


_POS = np.empty((D,), np.int32)
for _g in range(WG):
    for _k in range(16):
        _POS[32 * _g + 2 * _k] = 32 * _g + _k
        _POS[32 * _g + 2 * _k + 1] = 32 * _g + 16 + _k


def kernel(features, nodes, to_neighs):
    del nodes
    B = to_neighs.shape[0]
    V = features.shape[0]
    tn = to_neighs.astype(jnp.int32)
    packed = lax.bitcast_convert_type(
        features.astype(jnp.bfloat16).reshape(V, DPK, 2), jnp.float32)
    bp_unit = NW * NODES_PER_CHUNK * NBUF
    BP = ((B + bp_unit - 1) // bp_unit) * bp_unit
    b_per_w = BP // NW
    nchunk = b_per_w * DEG // GROW
    if BP != B:
        tn = jnp.pad(tn, ((0, BP - B), (0, 0)))
    idx_all = tn.reshape(NW, nchunk, GROW)

    mesh = plsc.VectorSubcoreMesh(core_axis_name="c", subcore_axis_name="s")
    run = pl.kernel(
        functools.partial(_agg_body, b_per_w, nchunk),
        out_type=jax.ShapeDtypeStruct((BP, D), jnp.float32),
        mesh=mesh,
        compiler_params=pltpu.CompilerParams(use_tc_tiling_on_sc=False),
        scratch_types=(
            [pltpu.VMEM((nchunk, GROW), jnp.int32)]
            + [pltpu.VMEM((b_per_w, D), jnp.float32)]
            + [pltpu.VMEM((GROW, DPK), jnp.float32) for _ in range(NBUF)]
            + [pltpu.SemaphoreType.DMA for _ in range(NBUF)]
        ),
    )
    out = run(packed, idx_all)
    return jnp.take(out[:B], jnp.asarray(_POS), axis=1)


# speedup vs baseline: 1.7234x; 1.7234x over previous
import functools

import jax
import jax.numpy as jnp
from jax import lax
from jax.experimental import pallas as pl
from jax.experimental.pallas import tpu as pltpu
from jax.experimental.pallas import tpu_sc as plsc

NC = 2
NS = 16
NW = NC * NS
DEG = 32
D = 128
GROW = 128
NODES_PER_CHUNK = GROW // DEG
DCH = D // 16
NBUF = 4


def _agg_body(b_per_w, nchunk, features, idx_all, out, idx_v, acc_v,
              *scratch):
    bufs = scratch[:NBUF]
    sems = scratch[NBUF:]
    wid = lax.axis_index("s") * NC + lax.axis_index("c")
    pltpu.sync_copy(idx_all.at[wid], idx_v)
    for b in range(NBUF):
        pltpu.async_copy(features.at[idx_v.at[b]], bufs[b], sems[b])

    def compute_chunk(c, buf):
        def node_body(n, carry):
            row0 = n * DEG
            for dc in range(DCH):
                a = buf[row0, pl.ds(dc * 16, 16)]
                for j in range(1, DEG):
                    a = a + buf[row0 + j, pl.ds(dc * 16, 16)]
                acc_v[c * NODES_PER_CHUNK + n, pl.ds(dc * 16, 16)] = a
            return carry
        lax.fori_loop(0, NODES_PER_CHUNK, node_body, 0)

    def group_body(i, carry):
        c_base = NBUF * i
        for b in range(NBUF):
            c = c_base + b
            pltpu.make_async_copy(features.at[idx_v.at[c]], bufs[b],
                                  sems[b]).wait()
            compute_chunk(c, bufs[b])

            @pl.when(c + NBUF < nchunk)
            def _():
                pltpu.async_copy(features.at[idx_v.at[c + NBUF]], bufs[b],
                                 sems[b])

        return carry

    lax.fori_loop(0, nchunk // NBUF, group_body, 0)
    pltpu.sync_copy(acc_v, out.at[pl.ds(wid * b_per_w, b_per_w)])


def kernel(features, nodes, to_neighs):
    del nodes
    B = to_neighs.shape[0]
    tn = to_neighs.astype(jnp.int32)
    bp_unit = NW * NODES_PER_CHUNK * NBUF
    BP = ((B + bp_unit - 1) // bp_unit) * bp_unit
    b_per_w = BP // NW
    nchunk = b_per_w * DEG // GROW
    if BP != B:
        tn = jnp.pad(tn, ((0, BP - B), (0, 0)))
    idx_all = tn.reshape(NW, nchunk, GROW)

    mesh = plsc.VectorSubcoreMesh(core_axis_name="c", subcore_axis_name="s")
    run = pl.kernel(
        functools.partial(_agg_body, b_per_w, nchunk),
        out_type=jax.ShapeDtypeStruct((BP, D), jnp.float32),
        mesh=mesh,
        scratch_types=(
            [pltpu.VMEM((nchunk, GROW), jnp.int32)]
            + [pltpu.VMEM((b_per_w, D), jnp.float32)]
            + [pltpu.VMEM((GROW, D), jnp.float32) for _ in range(NBUF)]
            + [pltpu.SemaphoreType.DMA for _ in range(NBUF)]
        ),
    )
    out = run(features, idx_all)
    return out[:B]
